# Initial kernel scaffold; baseline (speedup 1.0000x reference)
import functools

import jax
import jax.numpy as jnp
from jax import lax
from jax.experimental import pallas as pl
from jax.experimental.pallas import tpu as pltpu
from jax.experimental.pallas import tpu_sc as plsc

_BN_EPS = 1e-5
_NC = 2
_NS = 16
_C = 80


def _make_agg(n, e, d):
    nw = _NC * _NS
    ep = e // nw
    nch = ep // _C
    rpt = n // _NS
    assert ep % _C == 0 and n % _NS == 0

    mesh = plsc.VectorSubcoreMesh(core_axis_name="c", subcore_axis_name="s")

    @functools.partial(
        pl.kernel,
        out_type=jax.ShapeDtypeStruct((_NC, n, d), jnp.float32),
        mesh=mesh,
        scratch_types=[
            pltpu.VMEM((nch, _C), jnp.int32),
            pltpu.VMEM((nch, _C), jnp.int32),
            pltpu.VMEM((_C, d), jnp.float32),
            pltpu.VMEM_SHARED((n, d), jnp.float32),
            pltpu.SemaphoreType.DMA,
        ],
    )
    def agg(h_hbm, src_hbm, dst_hbm, zeros_hbm, parts_hbm,
            src_t, dst_t, rows, acc_sh, sem):
        c = lax.axis_index("c")
        s = lax.axis_index("s")
        wid = c * _NS + s
        pltpu.sync_copy(src_hbm.at[wid], src_t)
        pltpu.sync_copy(dst_hbm.at[wid], dst_t)
        pltpu.sync_copy(zeros_hbm, acc_sh.at[pl.ds(s * rpt, rpt)])
        plsc.subcore_barrier()

        def step(i, carry):
            pltpu.async_copy(h_hbm.at[src_t.at[i]], rows, sem).wait()
            pltpu.sync_copy(rows, acc_sh.at[dst_t.at[i]], add=True)
            return carry

        lax.fori_loop(0, nch, step, 0)
        plsc.subcore_barrier()
        pltpu.sync_copy(acc_sh.at[pl.ds(s * rpt, rpt)],
                        parts_hbm.at[c, pl.ds(s * rpt, rpt)])

    return agg


def _mm_body(h, parts, w, b, out, stats):
    z = h[...] + parts[0] + parts[1]
    y = jnp.dot(z, w[...], preferred_element_type=jnp.float32) + b[...]
    y = jnp.maximum(y, 0.0)
    out[...] = y

    @pl.when(pl.program_id(0) == 0)
    def _():
        stats[...] = jnp.zeros_like(stats)

    stats[0:1, :] += jnp.sum(y, axis=0, keepdims=True)
    stats[1:2, :] += jnp.sum(y * y, axis=0, keepdims=True)


def _make_mm(n, d, h_dim, rb):
    nb = n // rb
    return pl.pallas_call(
        _mm_body,
        grid=(nb,),
        in_specs=[
            pl.BlockSpec((rb, d), lambda i: (i, 0)),
            pl.BlockSpec((_NC, rb, d), lambda i: (0, i, 0)),
            pl.BlockSpec((d, h_dim), lambda i: (0, 0)),
            pl.BlockSpec((1, h_dim), lambda i: (0, 0)),
        ],
        out_specs=[
            pl.BlockSpec((rb, h_dim), lambda i: (i, 0)),
            pl.BlockSpec((8, h_dim), lambda i: (0, 0)),
        ],
        out_shape=[
            jax.ShapeDtypeStruct((n, h_dim), jnp.float32),
            jax.ShapeDtypeStruct((8, h_dim), jnp.float32),
        ],
    )


def _make_norm(n, h_dim, rb):
    nb = n // rb

    def body(y, stats, gamma, beta, out):
        mean = stats[0:1, :] * (1.0 / n)
        var = stats[1:2, :] * (1.0 / n) - mean * mean
        inv = gamma[...] * lax.rsqrt(var + _BN_EPS)
        out[...] = (y[...] - mean) * inv + beta[...]

    return pl.pallas_call(
        body,
        grid=(nb,),
        in_specs=[
            pl.BlockSpec((rb, h_dim), lambda i: (i, 0)),
            pl.BlockSpec((8, h_dim), lambda i: (0, 0)),
            pl.BlockSpec((1, h_dim), lambda i: (0, 0)),
            pl.BlockSpec((1, h_dim), lambda i: (0, 0)),
        ],
        out_specs=pl.BlockSpec((rb, h_dim), lambda i: (i, 0)),
        out_shape=jax.ShapeDtypeStruct((n, h_dim), jnp.float32),
    )


def kernel(x, edge_index, W1, b1, gamma1, beta1, W2, b2, gamma2, beta2):
    n, d = x.shape
    e = edge_index.shape[1]
    h_dim = W1.shape[1]
    nw = _NC * _NS
    ep = e // nw

    agg = _make_agg(n, e, d)
    mm = _make_mm(n, d, h_dim, rb=1000)
    norm = _make_norm(n, h_dim, rb=1000)

    src3 = edge_index[0].reshape(nw, ep // _C, _C)
    dst3 = edge_index[1].reshape(nw, ep // _C, _C)
    zeros = jnp.zeros((n // _NS, d), jnp.float32)

    def layer(h, w, b, gamma, beta):
        parts = agg(h, src3, dst3, zeros)
        y, stats = mm(h, parts, w, b.reshape(1, h_dim))
        return norm(y, stats, gamma.reshape(1, h_dim), beta.reshape(1, h_dim))

    h1 = layer(x, W1, b1, gamma1, beta1)
    return layer(h1, W2, b2, gamma2, beta2)


# speedup vs baseline: 7.0609x; 7.0609x over previous
import functools

import jax
import jax.numpy as jnp
from jax import lax
from jax.experimental import pallas as pl
from jax.experimental.pallas import tpu as pltpu
from jax.experimental.pallas import tpu_sc as plsc

_BN_EPS = 1e-5
_NC = 2
_NS = 16
_C = 80


def _make_agg(n_pad, e, d):
    nw = _NC * _NS
    ep = e // nw
    nch = ep // _C
    rpt = n_pad // _NS
    assert ep % _C == 0 and n_pad % (_NS * 8) == 0

    mesh = plsc.VectorSubcoreMesh(core_axis_name="c", subcore_axis_name="s")

    @functools.partial(
        pl.kernel,
        out_type=jax.ShapeDtypeStruct((_NC, n_pad, d), jnp.float32),
        mesh=mesh,
        scratch_types=[
            pltpu.VMEM((nch, _C), jnp.int32),
            pltpu.VMEM((nch, _C), jnp.int32),
            pltpu.VMEM((_C, d), jnp.float32),
            pltpu.VMEM_SHARED((n_pad, d), jnp.float32),
            pltpu.SemaphoreType.DMA,
        ],
    )
    def agg(h_hbm, src_hbm, dst_hbm, zeros_hbm, parts_hbm,
            src_t, dst_t, rows, acc_sh, sem):
        c = lax.axis_index("c")
        s = lax.axis_index("s")
        wid = c * _NS + s
        pltpu.sync_copy(src_hbm.at[wid], src_t)
        pltpu.sync_copy(dst_hbm.at[wid], dst_t)
        pltpu.sync_copy(zeros_hbm, acc_sh.at[pl.ds(s * rpt, rpt)])
        plsc.subcore_barrier()

        def step(i, carry):
            pltpu.async_copy(h_hbm.at[src_t.at[i]], rows, sem).wait()
            pltpu.sync_copy(rows, acc_sh.at[dst_t.at[i]], add=True)
            return carry

        lax.fori_loop(0, nch, step, 0)
        plsc.subcore_barrier()
        pltpu.sync_copy(acc_sh.at[pl.ds(s * rpt, rpt)],
                        parts_hbm.at[c, pl.ds(s * rpt, rpt)])

    return agg


def _mm_body(h, parts, w, b, out, stats):
    z = h[...] + parts[0] + parts[1]
    y = jnp.dot(z, w[...], preferred_element_type=jnp.float32) + b[...]
    y = jnp.maximum(y, 0.0)
    out[...] = y

    @pl.when(pl.program_id(0) == 0)
    def _():
        stats[...] = jnp.zeros_like(stats)

    stats[0:1, :] += jnp.sum(y, axis=0, keepdims=True)
    stats[1:2, :] += jnp.sum(y * y, axis=0, keepdims=True)


def _make_mm(n, d, h_dim, rb):
    nb = n // rb
    return pl.pallas_call(
        _mm_body,
        grid=(nb,),
        in_specs=[
            pl.BlockSpec((rb, d), lambda i: (i, 0)),
            pl.BlockSpec((_NC, rb, d), lambda i: (0, i, 0)),
            pl.BlockSpec((d, h_dim), lambda i: (0, 0)),
            pl.BlockSpec((1, h_dim), lambda i: (0, 0)),
        ],
        out_specs=[
            pl.BlockSpec((rb, h_dim), lambda i: (i, 0)),
            pl.BlockSpec((8, h_dim), lambda i: (0, 0)),
        ],
        out_shape=[
            jax.ShapeDtypeStruct((n, h_dim), jnp.float32),
            jax.ShapeDtypeStruct((8, h_dim), jnp.float32),
        ],
    )


def _make_norm(n, h_dim, rb):
    nb = n // rb

    def body(y, stats, gamma, beta, out):
        mean = stats[0:1, :] * (1.0 / n)
        var = stats[1:2, :] * (1.0 / n) - mean * mean
        inv = gamma[...] * lax.rsqrt(var + _BN_EPS)
        out[...] = (y[...] - mean) * inv + beta[...]

    return pl.pallas_call(
        body,
        grid=(nb,),
        in_specs=[
            pl.BlockSpec((rb, h_dim), lambda i: (i, 0)),
            pl.BlockSpec((8, h_dim), lambda i: (0, 0)),
            pl.BlockSpec((1, h_dim), lambda i: (0, 0)),
            pl.BlockSpec((1, h_dim), lambda i: (0, 0)),
        ],
        out_specs=pl.BlockSpec((rb, h_dim), lambda i: (i, 0)),
        out_shape=jax.ShapeDtypeStruct((n, h_dim), jnp.float32),
    )


def kernel(x, edge_index, W1, b1, gamma1, beta1, W2, b2, gamma2, beta2):
    n, d = x.shape
    e = edge_index.shape[1]
    h_dim = W1.shape[1]
    nw = _NC * _NS
    ep = e // nw
    n_pad = ((n + _NS * 8 - 1) // (_NS * 8)) * (_NS * 8)

    agg = _make_agg(n_pad, e, d)
    mm = _make_mm(n, d, h_dim, rb=1000)
    norm = _make_norm(n, h_dim, rb=1000)

    src3 = edge_index[0].reshape(nw, ep // _C, _C)
    dst3 = edge_index[1].reshape(nw, ep // _C, _C)
    zeros = jnp.zeros((n_pad // _NS, d), jnp.float32)

    def layer(h, w, b, gamma, beta):
        parts = agg(h, src3, dst3, zeros)
        y, stats = mm(h, parts, w, b.reshape(1, h_dim))
        return norm(y, stats, gamma.reshape(1, h_dim), beta.reshape(1, h_dim))

    h1 = layer(x, W1, b1, gamma1, beta1)
    return layer(h1, W2, b2, gamma2, beta2)
